# Initial kernel scaffold; baseline (speedup 1.0000x reference)
import jax
import jax.numpy as jnp
from jax.experimental import pallas as pl


def kernel(x, landmarks, cts_emb_landmarks, cat_tables):
    raise NotImplementedError("write your pallas kernel here")


# speedup vs baseline: 1.8315x; 1.8315x over previous
import functools

import jax
import jax.numpy as jnp
from jax import lax
from jax.experimental import pallas as pl
from jax.experimental.pallas import tpu as pltpu
from jax.experimental.pallas import tpu_sc as plsc

M = 32
B = 16384
N_CTS = 4
N_CAT = 26
LEVELS = 100000

NC = 2
NS = 16
NW = NC * NS

IPW = B * N_CAT // NW
CPW = IPW // 128
GROUP = 8
NGROUP = CPW // GROUP


def _cat_body(table_hbm, idx_hbm, out_hbm, idx_v, stage_v, sem):
    wid = lax.axis_index("s") * NC + lax.axis_index("c")
    crow = wid * CPW
    pltpu.sync_copy(idx_hbm.at[pl.ds(crow, CPW)], idx_v)

    def group(g, carry):
        handles = []
        for b in range(GROUP):
            c = g * GROUP + b
            handles.append(
                pltpu.async_copy(
                    table_hbm.at[idx_v.at[c]],
                    stage_v.at[pl.ds(b * 128, 128)],
                    sem,
                )
            )
        for h in handles:
            h.wait()
        pltpu.sync_copy(
            stage_v, out_hbm.at[pl.ds((crow + g * GROUP) * 128, GROUP * 128)]
        )
        return carry

    lax.fori_loop(0, NGROUP, group, 0)


@functools.partial(
    pl.kernel,
    out_type=jax.ShapeDtypeStruct((B * N_CAT, M), jnp.float32),
    mesh=plsc.VectorSubcoreMesh(core_axis_name="c", subcore_axis_name="s"),
    compiler_params=pltpu.CompilerParams(use_tc_tiling_on_sc=False),
    scratch_types=[
        pltpu.VMEM((CPW, 128), jnp.int32),
        pltpu.VMEM((GROUP * 128, M), jnp.float32),
        pltpu.SemaphoreType.DMA,
    ],
)
def _cat_gather(table_hbm, idx_hbm, out_hbm, idx_v, stage_v, sem):
    _cat_body(table_hbm, idx_hbm, out_hbm, idx_v, stage_v, sem)


BC = 2048


def _cts_body(xc_ref, lm_ref, emb_ref, out_ref):
    xc = xc_ref[...]
    for i in range(N_CTS):
        xv = xc[:, i]
        lm = lm_ref[i]
        indx = jnp.sum((lm[None, :] < xv[:, None]).astype(jnp.int32), axis=1)
        indx = jnp.clip(indx, 1, M - 1)
        cols = lax.broadcasted_iota(jnp.int32, (BC, M), 1)
        oh1 = (cols == indx[:, None]).astype(jnp.float32)
        oh0 = (cols == indx[:, None] - 1).astype(jnp.float32)
        lm1 = jnp.sum(oh1 * lm[None, :], axis=1)
        lm0 = jnp.sum(oh0 * lm[None, :], axis=1)
        t = (xv - lm0) / (lm1 - lm0)
        w = oh0 * (1.0 - t)[:, None] + oh1 * t[:, None]
        out_ref[:, i, :] = lax.dot(
            w, emb_ref[i], precision=lax.Precision.HIGHEST,
            preferred_element_type=jnp.float32,
        )


_cts_call = pl.pallas_call(
    _cts_body,
    grid=(B // BC,),
    in_specs=[
        pl.BlockSpec((BC, N_CTS), lambda i: (i, 0)),
        pl.BlockSpec((N_CTS, M), lambda i: (0, 0)),
        pl.BlockSpec((N_CTS, M, M), lambda i: (0, 0, 0)),
    ],
    out_specs=pl.BlockSpec((BC, N_CTS, M), lambda i: (i, 0, 0)),
    out_shape=jax.ShapeDtypeStruct((B, N_CTS, M), jnp.float32),
)


def kernel(x, landmarks, cts_emb_landmarks, cat_tables):
    xc = x[:, :N_CTS].astype(jnp.float32)
    offs = (jnp.arange(N_CAT, dtype=jnp.int32) * LEVELS)[None, :]
    idx2d = (x[:, N_CTS:] + offs).reshape(B * N_CAT // 128, 128)
    table = cat_tables.reshape(N_CAT * LEVELS, M)

    out_cat = _cat_gather(table, idx2d)
    out_cts = _cts_call(xc, landmarks, cts_emb_landmarks)
    return jnp.concatenate([out_cts, out_cat.reshape(B, N_CAT, M)], axis=1)


# speedup vs baseline: 1.8946x; 1.0344x over previous
import functools

import jax
import jax.numpy as jnp
from jax import lax
from jax.experimental import pallas as pl
from jax.experimental.pallas import tpu as pltpu
from jax.experimental.pallas import tpu_sc as plsc

M = 32
B = 16384
N_CTS = 4
N_CAT = 26
LEVELS = 100000
N_FIELDS = N_CTS + N_CAT

NC = 2
NS = 16
NW = NC * NS

IPW = B * N_CAT // NW
CPW = IPW // 128
GROUP = 8
NGROUP = CPW // GROUP
CTS_PER_W = B * N_CTS // NW
CTS_CHUNKS = CTS_PER_W // (GROUP * 128)


def _sc_body(table, ctsin, idx_h, dstc_h, dstx_h, out, idx_v, dst_v,
             dstx_v, stage, sem_g, sem_s):
    wid = lax.axis_index("s") * NC + lax.axis_index("c")

    pltpu.sync_copy(dstx_h.at[pl.ds(wid * (CTS_PER_W // 128), CTS_PER_W // 128)],
                    dstx_v)
    for c in range(CTS_CHUNKS):
        pltpu.sync_copy(
            ctsin.at[pl.ds(wid * CTS_PER_W + c * GROUP * 128, GROUP * 128)],
            stage)
        hs = [
            pltpu.async_copy(stage.at[pl.ds(s * 128, 128)],
                             out.at[dstx_v.at[c * GROUP + s]], sem_s)
            for s in range(GROUP)
        ]
        for h in hs:
            h.wait()

    crow = wid * CPW
    pltpu.sync_copy(idx_h.at[pl.ds(crow, CPW)], idx_v)
    pltpu.sync_copy(dstc_h.at[pl.ds(crow, CPW)], dst_v)

    def group(g, carry):
        hs = [
            pltpu.async_copy(table.at[idx_v.at[g * GROUP + b]],
                             stage.at[pl.ds(b * 128, 128)], sem_g)
            for b in range(GROUP)
        ]
        for h in hs:
            h.wait()
        ss = [
            pltpu.async_copy(stage.at[pl.ds(b * 128, 128)],
                             out.at[dst_v.at[g * GROUP + b]], sem_s)
            for b in range(GROUP)
        ]
        for s in ss:
            s.wait()
        return carry

    lax.fori_loop(0, NGROUP, group, 0)


@functools.partial(
    pl.kernel,
    out_type=jax.ShapeDtypeStruct((B * N_FIELDS, M), jnp.float32),
    mesh=plsc.VectorSubcoreMesh(core_axis_name="c", subcore_axis_name="s"),
    compiler_params=pltpu.CompilerParams(use_tc_tiling_on_sc=False),
    scratch_types=[
        pltpu.VMEM((CPW, 128), jnp.int32),
        pltpu.VMEM((CPW, 128), jnp.int32),
        pltpu.VMEM((CTS_PER_W // 128, 128), jnp.int32),
        pltpu.VMEM((GROUP * 128, M), jnp.float32),
        pltpu.SemaphoreType.DMA,
        pltpu.SemaphoreType.DMA,
    ],
)
def _sc_place(table, ctsin, idx_h, dstc_h, dstx_h, out, idx_v, dst_v,
              dstx_v, stage, sem_g, sem_s):
    _sc_body(table, ctsin, idx_h, dstc_h, dstx_h, out, idx_v, dst_v,
             dstx_v, stage, sem_g, sem_s)


BC = 2048


def _cts_body(xf_ref, lm_ref, emb_ref, out_ref):
    xv = xf_ref[0, 0, :]
    lm = lm_ref[0, 0]
    indx = jnp.sum((lm[None, :] < xv[:, None]).astype(jnp.int32), axis=1)
    indx = jnp.clip(indx, 1, M - 1)
    cols = lax.broadcasted_iota(jnp.int32, (BC, M), 1)
    oh1 = (cols == indx[:, None]).astype(jnp.float32)
    oh0 = (cols == indx[:, None] - 1).astype(jnp.float32)
    lm1 = jnp.sum(oh1 * lm[None, :], axis=1)
    lm0 = jnp.sum(oh0 * lm[None, :], axis=1)
    t = (xv - lm0) / (lm1 - lm0)
    w = oh0 * (1.0 - t)[:, None] + oh1 * t[:, None]
    out_ref[...] = lax.dot(
        w, emb_ref[0], precision=lax.Precision.HIGHEST,
        preferred_element_type=jnp.float32,
    )


_cts_call = pl.pallas_call(
    _cts_body,
    grid=(N_CTS, B // BC),
    in_specs=[
        pl.BlockSpec((1, 1, BC), lambda i, j: (i, 0, j)),
        pl.BlockSpec((1, 1, M), lambda i, j: (i, 0, 0)),
        pl.BlockSpec((1, M, M), lambda i, j: (i, 0, 0)),
    ],
    out_specs=pl.BlockSpec((BC, M), lambda i, j: (i * (B // BC) + j, 0)),
    out_shape=jax.ShapeDtypeStruct((B * N_CTS, M), jnp.float32),
)


def kernel(x, landmarks, cts_emb_landmarks, cat_tables):
    xft3 = x[:, :N_CTS].astype(jnp.float32).T.reshape(N_CTS, 1, B)
    lm3 = landmarks.reshape(N_CTS, 1, M)
    ctsin = _cts_call(xft3, lm3, cts_emb_landmarks)

    offs = (jnp.arange(N_CAT, dtype=jnp.int32) * LEVELS)[None, :]
    idx_cat = (x[:, N_CTS:] + offs).reshape(B * N_CAT // 128, 128)
    base = jnp.arange(B, dtype=jnp.int32) * N_FIELDS
    dst_cat = (base[:, None] + N_CTS
               + jnp.arange(N_CAT, dtype=jnp.int32)[None, :]
               ).reshape(B * N_CAT // 128, 128)
    dst_cts = (base[None, :] + jnp.arange(N_CTS, dtype=jnp.int32)[:, None]
               ).reshape(B * N_CTS // 128, 128)
    table = cat_tables.reshape(N_CAT * LEVELS, M)

    out = _sc_place(table, ctsin, idx_cat, dst_cat, dst_cts)
    return out.reshape(B, N_FIELDS, M)


# speedup vs baseline: 1.9014x; 1.0036x over previous
import functools

import jax
import jax.numpy as jnp
from jax import lax
from jax.experimental import pallas as pl
from jax.experimental.pallas import tpu as pltpu
from jax.experimental.pallas import tpu_sc as plsc

M = 32
B = 16384
N_CTS = 4
N_CAT = 26
LEVELS = 100000
N_FIELDS = N_CTS + N_CAT

NC = 2
NS = 16
NW = NC * NS

IPW = B * N_CAT // NW
CPW = IPW // 128
GROUP = 8
NGROUP = CPW // GROUP
CTS_PER_W = B * N_CTS // NW
CTS_CHUNKS = CTS_PER_W // (GROUP * 128)


def _sc_body(table3, ctsin, idx_h, dstc_h, dstx_h, out3, idx_v, dst_v,
             dstx_v, stage, sem_g, sem_s):
    wid = lax.axis_index("s") * NC + lax.axis_index("c")
    out = out3

    pltpu.sync_copy(dstx_h.at[pl.ds(wid * (CTS_PER_W // 128), CTS_PER_W // 128)],
                    dstx_v)
    for c in range(CTS_CHUNKS):
        pltpu.sync_copy(
            ctsin.at[pl.ds(wid * CTS_PER_W + c * GROUP * 128, GROUP * 128)],
            stage)
        hs = [
            pltpu.async_copy(stage.at[pl.ds(s * 128, 128)],
                             out.at[dstx_v.at[c * GROUP + s]], sem_s)
            for s in range(GROUP)
        ]
        for h in hs:
            h.wait()

    crow = wid * CPW
    pltpu.sync_copy(idx_h.at[pl.ds(crow, CPW)], idx_v)
    pltpu.sync_copy(dstc_h.at[pl.ds(crow, CPW)], dst_v)

    def group(g, carry):
        j = (crow // GROUP + g) // (B // (GROUP * 128))
        hs = [
            pltpu.async_copy(table3.at[j].at[idx_v.at[g * GROUP + b]],
                             stage.at[pl.ds(b * 128, 128)], sem_g)
            for b in range(GROUP)
        ]
        for h in hs:
            h.wait()
        ss = [
            pltpu.async_copy(stage.at[pl.ds(b * 128, 128)],
                             out.at[dst_v.at[g * GROUP + b]], sem_s)
            for b in range(GROUP)
        ]
        for s in ss:
            s.wait()
        return carry

    lax.fori_loop(0, NGROUP, group, 0)


@functools.partial(
    pl.kernel,
    out_type=jax.ShapeDtypeStruct((B * N_FIELDS, M), jnp.float32),
    mesh=plsc.VectorSubcoreMesh(core_axis_name="c", subcore_axis_name="s"),
    compiler_params=pltpu.CompilerParams(use_tc_tiling_on_sc=False),
    scratch_types=[
        pltpu.VMEM((CPW, 128), jnp.int32),
        pltpu.VMEM((CPW, 128), jnp.int32),
        pltpu.VMEM((CTS_PER_W // 128, 128), jnp.int32),
        pltpu.VMEM((GROUP * 128, M), jnp.float32),
        pltpu.SemaphoreType.DMA,
        pltpu.SemaphoreType.DMA,
    ],
)
def _sc_place(table3, ctsin, idx_h, dstc_h, dstx_h, out3, idx_v, dst_v,
              dstx_v, stage, sem_g, sem_s):
    _sc_body(table3, ctsin, idx_h, dstc_h, dstx_h, out3, idx_v, dst_v,
             dstx_v, stage, sem_g, sem_s)


BC = 2048


def _cts_body(xf_ref, lm_ref, emb_ref, out_ref):
    xv = xf_ref[0, 0, :]
    lm = lm_ref[0, 0]
    indx = jnp.sum((lm[None, :] < xv[:, None]).astype(jnp.int32), axis=1)
    indx = jnp.clip(indx, 1, M - 1)
    cols = lax.broadcasted_iota(jnp.int32, (BC, M), 1)
    oh1 = (cols == indx[:, None]).astype(jnp.float32)
    oh0 = (cols == indx[:, None] - 1).astype(jnp.float32)
    lm1 = jnp.sum(oh1 * lm[None, :], axis=1)
    lm0 = jnp.sum(oh0 * lm[None, :], axis=1)
    t = (xv - lm0) / (lm1 - lm0)
    w = oh0 * (1.0 - t)[:, None] + oh1 * t[:, None]
    out_ref[...] = lax.dot(
        w, emb_ref[0], precision=lax.Precision.HIGHEST,
        preferred_element_type=jnp.float32,
    )


_cts_call = pl.pallas_call(
    _cts_body,
    grid=(N_CTS, B // BC),
    in_specs=[
        pl.BlockSpec((1, 1, BC), lambda i, j: (i, 0, j)),
        pl.BlockSpec((1, 1, M), lambda i, j: (i, 0, 0)),
        pl.BlockSpec((1, M, M), lambda i, j: (i, 0, 0)),
    ],
    out_specs=pl.BlockSpec((BC, M), lambda i, j: (i * (B // BC) + j, 0)),
    out_shape=jax.ShapeDtypeStruct((B * N_CTS, M), jnp.float32),
)


def kernel(x, landmarks, cts_emb_landmarks, cat_tables):
    xft3 = x[:, :N_CTS].astype(jnp.float32).T.reshape(N_CTS, 1, B)
    lm3 = landmarks.reshape(N_CTS, 1, M)
    ctsin = _cts_call(xft3, lm3, cts_emb_landmarks)

    idx_cat = x[:, N_CTS:].T.reshape(B * N_CAT // 128, 128)
    base = jnp.arange(B, dtype=jnp.int32) * N_FIELDS
    dst_cat = (base[None, :]
               + (N_CTS + jnp.arange(N_CAT, dtype=jnp.int32))[:, None]
               ).reshape(B * N_CAT // 128, 128)
    dst_cts = (base[None, :] + jnp.arange(N_CTS, dtype=jnp.int32)[:, None]
               ).reshape(B * N_CTS // 128, 128)
    out = _sc_place(cat_tables, ctsin, idx_cat, dst_cat, dst_cts)
    return out.reshape(B, N_FIELDS, M)


# speedup vs baseline: 2.2185x; 1.1668x over previous
import functools

import jax
import jax.numpy as jnp
from jax import lax
from jax.experimental import pallas as pl
from jax.experimental.pallas import tpu as pltpu
from jax.experimental.pallas import tpu_sc as plsc

M = 32
B = 16384
N_CTS = 4
N_CAT = 26
LEVELS = 100000
N_FIELDS = N_CTS + N_CAT

NC = 2
NS = 16
NW = NC * NS

PAIRS = N_FIELDS * M
PPW = PAIRS // NW
BCH = 4096
NBCH = B // BCH


def _sc_body(table_t, idx_t, i0_t, t_t, emb_t, out, row_v, idx_v, col_v,
             ec_v, t_v):
    wid = lax.axis_index("s") * NC + lax.axis_index("c")

    def pair(p, carry):
        gp = wid * PPW + p
        f = gp // M
        d = gp % M

        @pl.when(f >= N_CTS)
        def _cat():
            j = f - N_CTS
            pltpu.sync_copy(table_t.at[j, d], row_v)

            def chunk(c, carry2):
                pltpu.sync_copy(idx_t.at[j, pl.ds(c * BCH, BCH)], idx_v)

                def vec(k, carry3):
                    g = plsc.load_gather(row_v, [idx_v[pl.ds(k * 16, 16)]])
                    col_v[pl.ds(k * 16, 16)] = g
                    return carry3

                lax.fori_loop(0, BCH // 16, vec, 0, unroll=8)
                pltpu.sync_copy(col_v, out.at[f, d, pl.ds(c * BCH, BCH)])
                return carry2

            lax.fori_loop(0, NBCH, chunk, 0)

        @pl.when(f < N_CTS)
        def _cts():
            pltpu.sync_copy(emb_t.at[f, d], ec_v)

            def chunk(c, carry2):
                pltpu.sync_copy(i0_t.at[f, pl.ds(c * BCH, BCH)], idx_v)
                pltpu.sync_copy(t_t.at[f, pl.ds(c * BCH, BCH)], t_v)

                def vec(k, carry3):
                    i016 = idx_v[pl.ds(k * 16, 16)]
                    t16 = t_v[pl.ds(k * 16, 16)]
                    e0 = plsc.load_gather(ec_v, [i016])
                    e1 = plsc.load_gather(ec_v, [i016 + 1])
                    col_v[pl.ds(k * 16, 16)] = e0 + t16 * (e1 - e0)
                    return carry3

                lax.fori_loop(0, BCH // 16, vec, 0, unroll=8)
                pltpu.sync_copy(col_v, out.at[f, d, pl.ds(c * BCH, BCH)])
                return carry2

            lax.fori_loop(0, NBCH, chunk, 0)

        return carry

    lax.fori_loop(0, PPW, pair, 0)


@functools.partial(
    pl.kernel,
    out_type=jax.ShapeDtypeStruct((N_FIELDS, M, B), jnp.float32),
    mesh=plsc.VectorSubcoreMesh(core_axis_name="c", subcore_axis_name="s"),
    compiler_params=pltpu.CompilerParams(
        use_tc_tiling_on_sc=False, needs_layout_passes=False),
    scratch_types=[
        pltpu.VMEM((LEVELS,), jnp.float32),
        pltpu.VMEM((BCH,), jnp.int32),
        pltpu.VMEM((BCH,), jnp.float32),
        pltpu.VMEM((M,), jnp.float32),
        pltpu.VMEM((BCH,), jnp.float32),
    ],
)
def _sc_cols(table_t, idx_t, i0_t, t_t, emb_t, out, row_v, idx_v, col_v,
             ec_v, t_v):
    _sc_body(table_t, idx_t, i0_t, t_t, emb_t, out, row_v, idx_v, col_v,
             ec_v, t_v)


BC = 2048


def _prep_body(xf_ref, lm_ref, i0_ref, t_ref):
    xv = xf_ref[0, 0, :]
    lm = lm_ref[0, 0]
    indx = jnp.sum((lm[None, :] < xv[:, None]).astype(jnp.int32), axis=1)
    indx = jnp.clip(indx, 1, M - 1)
    cols = lax.broadcasted_iota(jnp.int32, (BC, M), 1)
    oh1 = (cols == indx[:, None]).astype(jnp.float32)
    oh0 = (cols == indx[:, None] - 1).astype(jnp.float32)
    lm1 = jnp.sum(oh1 * lm[None, :], axis=1)
    lm0 = jnp.sum(oh0 * lm[None, :], axis=1)
    i0_ref[0, 0, :] = indx - 1
    t_ref[0, 0, :] = (xv - lm0) / (lm1 - lm0)


_prep_call = pl.pallas_call(
    _prep_body,
    grid=(N_CTS, B // BC),
    in_specs=[
        pl.BlockSpec((1, 1, BC), lambda i, j: (i, 0, j)),
        pl.BlockSpec((1, 1, M), lambda i, j: (i, 0, 0)),
    ],
    out_specs=[
        pl.BlockSpec((1, 1, BC), lambda i, j: (i, 0, j)),
        pl.BlockSpec((1, 1, BC), lambda i, j: (i, 0, j)),
    ],
    out_shape=[
        jax.ShapeDtypeStruct((N_CTS, 1, B), jnp.int32),
        jax.ShapeDtypeStruct((N_CTS, 1, B), jnp.float32),
    ],
)


def kernel(x, landmarks, cts_emb_landmarks, cat_tables):
    xft3 = x[:, :N_CTS].astype(jnp.float32).T.reshape(N_CTS, 1, B)
    lm3 = landmarks.reshape(N_CTS, 1, M)
    i03, t3 = _prep_call(xft3, lm3)

    table_t = cat_tables.transpose(0, 2, 1)
    emb_t = cts_emb_landmarks.transpose(0, 2, 1)
    idx_t = x[:, N_CTS:].T

    out_t = _sc_cols(table_t, idx_t, i03.reshape(N_CTS, B),
                     t3.reshape(N_CTS, B), emb_t)
    return out_t.transpose(2, 0, 1)


# speedup vs baseline: 3.7560x; 1.6930x over previous
import functools

import jax
import jax.numpy as jnp
from jax import lax
from jax.experimental import pallas as pl
from jax.experimental.pallas import tpu as pltpu
from jax.experimental.pallas import tpu_sc as plsc

M = 32
B = 16384
N_CTS = 4
N_CAT = 26
LEVELS = 100000
N_FIELDS = N_CTS + N_CAT

NC = 2
NS = 16
NW = NC * NS

PAIRS = N_FIELDS * M
PPW = PAIRS // NW
BCH = 4096
NBCH = B // BCH


def _sc_body(table_t, idx_t, i0_t, t_t, emb_t, out, row_v, idx_v, col_v,
             ec_v, t_v):
    wid = lax.axis_index("s") * NC + lax.axis_index("c")

    def pair(p, carry):
        gp = wid * PPW + p
        f = gp // M
        d = gp % M

        @pl.when(f >= N_CTS)
        def _cat():
            j = f - N_CTS
            pltpu.sync_copy(table_t.at[j, d], row_v)

            def chunk(c, carry2):
                pltpu.sync_copy(idx_t.at[j, pl.ds(c * BCH, BCH)], idx_v)

                def vec(k, carry3):
                    g = plsc.load_gather(row_v, [idx_v[pl.ds(k * 16, 16)]])
                    col_v[pl.ds(k * 16, 16)] = g
                    return carry3

                lax.fori_loop(0, BCH // 16, vec, 0, unroll=8)
                pltpu.sync_copy(col_v, out.at[f, d, pl.ds(c * BCH, BCH)])
                return carry2

            lax.fori_loop(0, NBCH, chunk, 0)

        @pl.when(f < N_CTS)
        def _cts():
            pltpu.sync_copy(emb_t.at[f, d], ec_v)

            def chunk(c, carry2):
                pltpu.sync_copy(i0_t.at[f, pl.ds(c * BCH, BCH)], idx_v)
                pltpu.sync_copy(t_t.at[f, pl.ds(c * BCH, BCH)], t_v)

                def vec(k, carry3):
                    i016 = idx_v[pl.ds(k * 16, 16)]
                    t16 = t_v[pl.ds(k * 16, 16)]
                    e0 = plsc.load_gather(ec_v, [i016])
                    e1 = plsc.load_gather(ec_v, [i016 + 1])
                    col_v[pl.ds(k * 16, 16)] = e0 + t16 * (e1 - e0)
                    return carry3

                lax.fori_loop(0, BCH // 16, vec, 0, unroll=8)
                pltpu.sync_copy(col_v, out.at[f, d, pl.ds(c * BCH, BCH)])
                return carry2

            lax.fori_loop(0, NBCH, chunk, 0)

        return carry

    lax.fori_loop(0, PPW, pair, 0)


@functools.partial(
    pl.kernel,
    out_type=jax.ShapeDtypeStruct((N_FIELDS, M, B), jnp.float32),
    mesh=plsc.VectorSubcoreMesh(core_axis_name="c", subcore_axis_name="s"),
    compiler_params=pltpu.CompilerParams(
        use_tc_tiling_on_sc=True, needs_layout_passes=False),
    scratch_types=[
        pltpu.VMEM((LEVELS,), jnp.float32),
        pltpu.VMEM((BCH,), jnp.int32),
        pltpu.VMEM((BCH,), jnp.float32),
        pltpu.VMEM((M,), jnp.float32),
        pltpu.VMEM((BCH,), jnp.float32),
    ],
)
def _sc_cols(table_t, idx_t, i0_t, t_t, emb_t, out, row_v, idx_v, col_v,
             ec_v, t_v):
    _sc_body(table_t, idx_t, i0_t, t_t, emb_t, out, row_v, idx_v, col_v,
             ec_v, t_v)


BC = 2048


def _prep_body(xf_ref, lm_ref, i0_ref, t_ref):
    xv = xf_ref[0, 0, :]
    lm = lm_ref[0, 0]
    indx = jnp.sum((lm[None, :] < xv[:, None]).astype(jnp.int32), axis=1)
    indx = jnp.clip(indx, 1, M - 1)
    cols = lax.broadcasted_iota(jnp.int32, (BC, M), 1)
    oh1 = (cols == indx[:, None]).astype(jnp.float32)
    oh0 = (cols == indx[:, None] - 1).astype(jnp.float32)
    lm1 = jnp.sum(oh1 * lm[None, :], axis=1)
    lm0 = jnp.sum(oh0 * lm[None, :], axis=1)
    i0_ref[0, 0, :] = indx - 1
    t_ref[0, 0, :] = (xv - lm0) / (lm1 - lm0)


_prep_call = pl.pallas_call(
    _prep_body,
    grid=(N_CTS, B // BC),
    in_specs=[
        pl.BlockSpec((1, 1, BC), lambda i, j: (i, 0, j)),
        pl.BlockSpec((1, 1, M), lambda i, j: (i, 0, 0)),
    ],
    out_specs=[
        pl.BlockSpec((1, 1, BC), lambda i, j: (i, 0, j)),
        pl.BlockSpec((1, 1, BC), lambda i, j: (i, 0, j)),
    ],
    out_shape=[
        jax.ShapeDtypeStruct((N_CTS, 1, B), jnp.int32),
        jax.ShapeDtypeStruct((N_CTS, 1, B), jnp.float32),
    ],
)


def kernel(x, landmarks, cts_emb_landmarks, cat_tables):
    xft3 = x[:, :N_CTS].astype(jnp.float32).T.reshape(N_CTS, 1, B)
    lm3 = landmarks.reshape(N_CTS, 1, M)
    i03, t3 = _prep_call(xft3, lm3)

    table_t = cat_tables.transpose(0, 2, 1)
    emb_t = cts_emb_landmarks.transpose(0, 2, 1)
    idx_t = x[:, N_CTS:].T

    out_t = _sc_cols(table_t, idx_t, i03.reshape(N_CTS, B),
                     t3.reshape(N_CTS, B), emb_t)
    return out_t.transpose(2, 0, 1)


# speedup vs baseline: 4.1974x; 1.1175x over previous
import functools

import jax
import jax.numpy as jnp
from jax import lax
from jax.experimental import pallas as pl
from jax.experimental.pallas import tpu as pltpu
from jax.experimental.pallas import tpu_sc as plsc

M = 32
B = 16384
N_CTS = 4
N_CAT = 26
LEVELS = 100000
N_FIELDS = N_CTS + N_CAT

NC = 2
NS = 16
NW = NC * NS

PAIRS = N_FIELDS * M
PPW = PAIRS // NW
BCH = 4096
NBCH = B // BCH


def _sc_body(table_t, idx_t, i0_t, t_t, emb_t, out, row_v, idx_v, col_a,
             col_b, t_v, ec_v, sem_w):
    wid = lax.axis_index("s") * NC + lax.axis_index("c")
    cols = [col_a, col_b]

    def write_chunks(f, d, gather_chunk):
        handles = [None, None]
        for c in range(NBCH):
            col = cols[c % 2]
            if handles[c % 2] is not None:
                handles[c % 2].wait()
            gather_chunk(c, col)
            handles[c % 2] = pltpu.async_copy(
                col, out.at[f, d, pl.ds(c * BCH, BCH)], sem_w)
        for h in handles:
            if h is not None:
                h.wait()

    def pair(p, jprev):
        gp = wid * PPW + p
        f = gp // M
        d = gp % M

        @pl.when(f >= N_CTS)
        def _cat():
            j = f - N_CTS

            @pl.when(j != jprev)
            def _stage_idx():
                pltpu.sync_copy(idx_t.at[j], idx_v)

            pltpu.sync_copy(table_t.at[j, d], row_v)

            def gather_chunk(c, col):
                def vec(k, carry3):
                    g = plsc.load_gather(
                        row_v, [idx_v[pl.ds(c * BCH + k * 16, 16)]])
                    col[pl.ds(k * 16, 16)] = g
                    return carry3

                lax.fori_loop(0, BCH // 16, vec, 0, unroll=8)

            write_chunks(f, d, gather_chunk)

        @pl.when(f < N_CTS)
        def _cts():
            pltpu.sync_copy(emb_t.at[f, d], ec_v)
            pltpu.sync_copy(i0_t.at[f], idx_v)

            def gather_chunk(c, col):
                pltpu.sync_copy(t_t.at[f, pl.ds(c * BCH, BCH)], t_v)

                def vec(k, carry3):
                    i016 = idx_v[pl.ds(c * BCH + k * 16, 16)]
                    t16 = t_v[pl.ds(k * 16, 16)]
                    e0 = plsc.load_gather(ec_v, [i016])
                    e1 = plsc.load_gather(ec_v, [i016 + 1])
                    col[pl.ds(k * 16, 16)] = e0 + t16 * (e1 - e0)
                    return carry3

                lax.fori_loop(0, BCH // 16, vec, 0, unroll=8)

            write_chunks(f, d, gather_chunk)

        return jnp.where(f >= N_CTS, f - N_CTS, -1)

    lax.fori_loop(0, PPW, pair, -1)


@functools.partial(
    pl.kernel,
    out_type=jax.ShapeDtypeStruct((N_FIELDS, M, B), jnp.float32),
    mesh=plsc.VectorSubcoreMesh(core_axis_name="c", subcore_axis_name="s"),
    compiler_params=pltpu.CompilerParams(
        use_tc_tiling_on_sc=True, needs_layout_passes=False),
    scratch_types=[
        pltpu.VMEM((LEVELS,), jnp.float32),
        pltpu.VMEM((B,), jnp.int32),
        pltpu.VMEM((BCH,), jnp.float32),
        pltpu.VMEM((BCH,), jnp.float32),
        pltpu.VMEM((BCH,), jnp.float32),
        pltpu.VMEM((M,), jnp.float32),
        pltpu.SemaphoreType.DMA,
    ],
)
def _sc_cols(table_t, idx_t, i0_t, t_t, emb_t, out, row_v, idx_v, col_a,
             col_b, t_v, ec_v, sem_w):
    _sc_body(table_t, idx_t, i0_t, t_t, emb_t, out, row_v, idx_v, col_a,
             col_b, t_v, ec_v, sem_w)


BC = 16384


def _prep_body(xf_ref, lm_ref, i0_ref, t_ref):
    xv = xf_ref[0, 0, :]
    lm = lm_ref[0, 0]
    indx = jnp.sum((lm[None, :] < xv[:, None]).astype(jnp.int32), axis=1)
    indx = jnp.clip(indx, 1, M - 1)
    cols = lax.broadcasted_iota(jnp.int32, (BC, M), 1)
    oh1 = (cols == indx[:, None]).astype(jnp.float32)
    oh0 = (cols == indx[:, None] - 1).astype(jnp.float32)
    lm1 = jnp.sum(oh1 * lm[None, :], axis=1)
    lm0 = jnp.sum(oh0 * lm[None, :], axis=1)
    i0_ref[0, 0, :] = indx - 1
    t_ref[0, 0, :] = (xv - lm0) / (lm1 - lm0)


_prep_call = pl.pallas_call(
    _prep_body,
    grid=(N_CTS, B // BC),
    in_specs=[
        pl.BlockSpec((1, 1, BC), lambda i, j: (i, 0, j)),
        pl.BlockSpec((1, 1, M), lambda i, j: (i, 0, 0)),
    ],
    out_specs=[
        pl.BlockSpec((1, 1, BC), lambda i, j: (i, 0, j)),
        pl.BlockSpec((1, 1, BC), lambda i, j: (i, 0, j)),
    ],
    out_shape=[
        jax.ShapeDtypeStruct((N_CTS, 1, B), jnp.int32),
        jax.ShapeDtypeStruct((N_CTS, 1, B), jnp.float32),
    ],
)


def kernel(x, landmarks, cts_emb_landmarks, cat_tables):
    xft3 = x[:, :N_CTS].astype(jnp.float32).T.reshape(N_CTS, 1, B)
    lm3 = landmarks.reshape(N_CTS, 1, M)
    i03, t3 = _prep_call(xft3, lm3)

    table_t = cat_tables.transpose(0, 2, 1)
    emb_t = cts_emb_landmarks.transpose(0, 2, 1)
    idx_t = x[:, N_CTS:].T

    out_t = _sc_cols(table_t, idx_t, i03.reshape(N_CTS, B),
                     t3.reshape(N_CTS, B), emb_t)
    return out_t.transpose(2, 0, 1)
